# baseline (speedup 1.0000x reference)
import functools

import jax
import jax.numpy as jnp
from jax import lax
from jax.experimental import pallas as pl
from jax.experimental.pallas import tpu as pltpu
from jax.experimental.pallas import tpu_sc as plsc

NUM_EXPERTS = 64
TOP_K = 8
HIDDEN = 4096
BLOCK_M = 512
ROWS = 16384
NW = 32
CHUNKS = 4
CROWS = ROWS // CHUNKS
RPW = CROWS // NW


def _logits_block(x_ref, w_ref, logits_ref):
    logits_ref[...] = jnp.dot(x_ref[...], w_ref[...],
                              preferred_element_type=jnp.float32)


def _merge16(a, ai, b, bi):
    br = lax.rev(b, (0,))
    bir = lax.rev(bi, (0,))
    take = a >= br
    m = jnp.where(take, a, br)
    mi = jnp.where(take, ai, bir)
    return plsc.sort_key_val(m, mi, descending=True)


def _sc_topk_body(logits_hbm, w_hbm, i_hbm, slab, wout, iout):
    wid = lax.axis_index("s") * 2 + lax.axis_index("c")
    base = wid * RPW
    pltpu.sync_copy(logits_hbm.at[pl.ds(base, RPW)], slab)

    lane = lax.iota(jnp.int32, 16)
    lane_lt8 = lane < TOP_K

    @plsc.parallel_loop(0, RPW, 1, unroll=4)
    def body(r):
        chunks = []
        for e in range(NUM_EXPERTS // 16):
            v = slab[r, pl.ds(e * 16, 16)]
            ii = lane + e * 16
            chunks.append(plsc.sort_key_val(v, ii, descending=True))
        m01 = _merge16(*chunks[0], *chunks[1])
        m23 = _merge16(*chunks[2], *chunks[3])
        t, ti = _merge16(*m01, *m23)

        ex = jnp.exp(t - jnp.max(t))
        ex8 = jnp.where(lane_lt8, ex, 0.0)
        w = ex8 / jnp.sum(ex8)

        row_idx = jnp.full((16,), r, jnp.int32)
        plsc.store_scatter(wout, [row_idx, lane], w, mask=lane_lt8)
        plsc.store_scatter(iout, [row_idx, lane], ti, mask=lane_lt8)

    pltpu.sync_copy(wout, w_hbm.at[pl.ds(base, RPW)])
    pltpu.sync_copy(iout, i_hbm.at[pl.ds(base, RPW)])


_sc_topk = functools.partial(
    pl.kernel,
    mesh=plsc.VectorSubcoreMesh(core_axis_name="c", subcore_axis_name="s"),
    compiler_params=pltpu.CompilerParams(needs_layout_passes=False,
                                         use_tc_tiling_on_sc=False),
    out_type=[
        jax.ShapeDtypeStruct((CROWS, TOP_K), jnp.float32),
        jax.ShapeDtypeStruct((CROWS, TOP_K), jnp.int32),
    ],
    scratch_types=[
        pltpu.VMEM((RPW, NUM_EXPERTS), jnp.float32),
        pltpu.VMEM((RPW, TOP_K), jnp.float32),
        pltpu.VMEM((RPW, TOP_K), jnp.int32),
    ],
)(_sc_topk_body)


@jax.jit
def kernel(hidden_states, weight):
    x = hidden_states.reshape(-1, HIDDEN)
    wt = weight.T
    tc_matmul = pl.pallas_call(
        _logits_block,
        grid=(CROWS // BLOCK_M,),
        in_specs=[
            pl.BlockSpec((BLOCK_M, HIDDEN), lambda i: (i, 0)),
            pl.BlockSpec((HIDDEN, NUM_EXPERTS), lambda i: (0, 0)),
        ],
        out_specs=pl.BlockSpec((BLOCK_M, NUM_EXPERTS), lambda i: (i, 0)),
        out_shape=jax.ShapeDtypeStruct((CROWS, NUM_EXPERTS), jnp.float32),
    )
    logits_c = [tc_matmul(x[c * CROWS:(c + 1) * CROWS], wt)
                for c in range(CHUNKS)]
    topk_c = [_sc_topk(lc) for lc in logits_c]
    logits = jnp.concatenate(logits_c, axis=0)
    weights = jnp.concatenate([t[0] for t in topk_c], axis=0)
    indices = jnp.concatenate([t[1] for t in topk_c], axis=0)
    return logits, weights, indices


# speedup vs baseline: 1.9972x; 1.9972x over previous
import functools

import jax
import jax.numpy as jnp
from jax import lax
from jax.experimental import pallas as pl
from jax.experimental.pallas import tpu as pltpu
from jax.experimental.pallas import tpu_sc as plsc

NUM_EXPERTS = 64
TOP_K = 8
HIDDEN = 4096
BLOCK_M = 512
ROWS = 16384
NW = 32
CHUNKS = 4
CROWS = ROWS // CHUNKS
RPW = CROWS // NW


def _logits_block(x_ref, w_ref, logits_ref):
    logits_ref[...] = jnp.dot(x_ref[...], w_ref[...],
                              preferred_element_type=jnp.float32)


def _merge16(a, ai, b, bi):
    br = lax.rev(b, (0,))
    bir = lax.rev(bi, (0,))
    take = a >= br
    m = jnp.where(take, a, br)
    mi = jnp.where(take, ai, bir)
    return plsc.sort_key_val(m, mi, descending=True)


def _sc_topk_body(logits_hbm, w_hbm, i_hbm, slab, wout, iout):
    wid = lax.axis_index("s") * 2 + lax.axis_index("c")
    base = wid * RPW
    pltpu.sync_copy(logits_hbm.at[pl.ds(base, RPW)], slab)

    lane = lax.iota(jnp.int32, 16)
    lane_lt8 = lane < TOP_K

    @plsc.parallel_loop(0, RPW, 1, unroll=4)
    def body(r):
        chunks = []
        for e in range(NUM_EXPERTS // 16):
            v = slab[r, pl.ds(e * 16, 16)]
            ii = lane + e * 16
            chunks.append(plsc.sort_key_val(v, ii, descending=True))
        m01 = _merge16(*chunks[0], *chunks[1])
        m23 = _merge16(*chunks[2], *chunks[3])
        t, ti = _merge16(*m01, *m23)

        ex = jnp.exp(t - jnp.max(t))
        ex8 = jnp.where(lane_lt8, ex, 0.0)
        w = ex8 / jnp.sum(ex8)

        row_idx = jnp.full((16,), r, jnp.int32)
        plsc.store_scatter(wout, [row_idx, lane], w, mask=lane_lt8)
        plsc.store_scatter(iout, [row_idx, lane], ti, mask=lane_lt8)

    pltpu.sync_copy(wout, w_hbm.at[pl.ds(base, RPW)])
    pltpu.sync_copy(iout, i_hbm.at[pl.ds(base, RPW)])


_sc_topk = functools.partial(
    pl.kernel,
    mesh=plsc.VectorSubcoreMesh(core_axis_name="c", subcore_axis_name="s"),
    compiler_params=pltpu.CompilerParams(needs_layout_passes=False,
                                         use_tc_tiling_on_sc=False),
    out_type=[
        jax.ShapeDtypeStruct((CROWS, TOP_K), jnp.float32),
        jax.ShapeDtypeStruct((CROWS, TOP_K), jnp.int32),
    ],
    scratch_types=[
        pltpu.VMEM((RPW, NUM_EXPERTS), jnp.float32),
        pltpu.VMEM((RPW, TOP_K), jnp.float32),
        pltpu.VMEM((RPW, TOP_K), jnp.int32),
    ],
)(_sc_topk_body)


@jax.jit
def kernel(hidden_states, weight):
    x = hidden_states.reshape(-1, HIDDEN)
    wt = weight.T
    blocks_per_chunk = CROWS // BLOCK_M

    def tc_matmul(c):
        return pl.pallas_call(
            _logits_block,
            grid=(blocks_per_chunk,),
            in_specs=[
                pl.BlockSpec((BLOCK_M, HIDDEN),
                             lambda i, c=c: (c * blocks_per_chunk + i, 0)),
                pl.BlockSpec((HIDDEN, NUM_EXPERTS), lambda i: (0, 0)),
            ],
            out_specs=pl.BlockSpec((BLOCK_M, NUM_EXPERTS), lambda i: (i, 0)),
            out_shape=jax.ShapeDtypeStruct((CROWS, NUM_EXPERTS),
                                           jnp.float32),
        )(x, wt)

    logits_c = [tc_matmul(c) for c in range(CHUNKS)]
    topk_c = [_sc_topk(lc) for lc in logits_c]
    logits = jnp.concatenate(logits_c, axis=0)
    weights = jnp.concatenate([t[0] for t in topk_c], axis=0)
    indices = jnp.concatenate([t[1] for t in topk_c], axis=0)
    return logits, weights, indices


# speedup vs baseline: 2.5302x; 1.2669x over previous
import functools

import jax
import jax.numpy as jnp
from jax.experimental import pallas as pl
from jax.experimental.pallas import tpu as pltpu

NUM_EXPERTS = 64
TOP_K = 8
HIDDEN = 4096
BLOCK_M = 1024


def _router_block(x_ref, w_ref, logits_ref, weights_ref, indices_ref):
    x = x_ref[...]
    w = w_ref[...]
    logits = jnp.dot(x, w, preferred_element_type=jnp.float32)
    logits_ref[...] = logits

    m = jnp.max(logits, axis=-1, keepdims=True)
    e = jnp.exp(logits - m)
    probs = e / jnp.sum(e, axis=-1, keepdims=True)

    col = jax.lax.broadcasted_iota(jnp.int32, probs.shape, 1)
    work = probs
    vals = []
    idxs = []
    for _ in range(TOP_K):
        mj = jnp.max(work, axis=-1, keepdims=True)
        ij = jnp.min(jnp.where(work == mj, col, NUM_EXPERTS), axis=-1,
                     keepdims=True)
        vals.append(mj)
        idxs.append(ij)
        work = jnp.where(col == ij, -1.0, work)

    top_vals = jnp.concatenate(vals, axis=-1)
    weights_ref[...] = top_vals / jnp.sum(top_vals, axis=-1, keepdims=True)
    indices_ref[...] = jnp.concatenate(idxs, axis=-1)


@jax.jit
def kernel(hidden_states, weight):
    x = hidden_states.reshape(-1, HIDDEN)
    rows = x.shape[0]
    wt = weight.T
    grid = (rows // BLOCK_M,)
    logits, weights, indices = pl.pallas_call(
        _router_block,
        grid=grid,
        in_specs=[
            pl.BlockSpec((BLOCK_M, HIDDEN), lambda i: (i, 0)),
            pl.BlockSpec((HIDDEN, NUM_EXPERTS), lambda i: (0, 0)),
        ],
        out_specs=[
            pl.BlockSpec((BLOCK_M, NUM_EXPERTS), lambda i: (i, 0)),
            pl.BlockSpec((BLOCK_M, TOP_K), lambda i: (i, 0)),
            pl.BlockSpec((BLOCK_M, TOP_K), lambda i: (i, 0)),
        ],
        out_shape=[
            jax.ShapeDtypeStruct((rows, NUM_EXPERTS), jnp.float32),
            jax.ShapeDtypeStruct((rows, TOP_K), jnp.float32),
            jax.ShapeDtypeStruct((rows, TOP_K), jnp.int32),
        ],
    )(x, wt)
    return logits, weights, indices
